# baseline (speedup 1.0000x reference)
import functools
import math

import jax
import jax.numpy as jnp
from jax import lax
from jax.experimental import pallas as pl
from jax.experimental.pallas import tpu as pltpu
from jax.experimental.pallas import tpu_sc as plsc

_TH2 = (0.75 * math.pi) ** 2
_K_SC = 2048


def _make_sc_kernel(K):
    info = plsc.get_sparse_core_info()
    NC, NS, L = info.num_cores, info.num_subcores, info.num_lanes
    NW = NC * NS
    tiles_per_w = K // NW
    CH = tiles_per_w
    mesh = plsc.VectorSubcoreMesh(core_axis_name="c", subcore_axis_name="s")

    @functools.partial(
        pl.kernel,
        out_type=jax.ShapeDtypeStruct((K * 128,), jnp.int32),
        mesh=mesh,
        scratch_types=[
            pltpu.VMEM((CH, 128), jnp.float32),
            pltpu.VMEM((CH, 128), jnp.float32),
            pltpu.VMEM((CH, 128), jnp.float32),
            pltpu.VMEM((CH * 128,), jnp.int32),
        ],
        compiler_params=pltpu.CompilerParams(needs_layout_passes=False),
    )
    def body(v_hbm, out_hbm, b0, b1, b2, obuf):
        wid = lax.axis_index("s") * NC + lax.axis_index("c")
        tc0 = wid * tiles_per_w
        pltpu.sync_copy(v_hbm.at[pl.ds(tc0, CH), 0, :], b0)
        pltpu.sync_copy(v_hbm.at[pl.ds(tc0, CH), 1, :], b1)
        pltpu.sync_copy(v_hbm.at[pl.ds(tc0, CH), 2, :], b2)

        @plsc.parallel_loop(0, CH * 8, unroll=8)
        def inner(i):
            j = i >> 3
            l0 = (i & 7) * L
            v0 = b0[j, pl.ds(l0, L)]
            v1 = b1[j, pl.ds(l0, L)]
            v2 = b2[j, pl.ds(l0, L)]
            s = v0 * v0 + v1 * v1 + v2 * v2
            obuf[pl.ds(i * L, L)] = (s > _TH2).astype(jnp.int32)

        pltpu.sync_copy(obuf, out_hbm.at[pl.ds(tc0 * 128, CH * 128)])

    return body


def _tc_body(v_hbm, o_ref, b0, b1, b2, sem):
    i = pl.program_id(0)
    blk = o_ref.shape[0]
    r0 = _K_SC + i * blk
    c0 = pltpu.make_async_copy(v_hbm.at[pl.ds(r0, blk), 0, :], b0, sem)
    c1 = pltpu.make_async_copy(v_hbm.at[pl.ds(r0, blk), 1, :], b1, sem)
    c2 = pltpu.make_async_copy(v_hbm.at[pl.ds(r0, blk), 2, :], b2, sem)
    c0.start()
    c1.start()
    c2.start()
    c0.wait()
    c1.wait()
    c2.wait()
    v0 = b0[...]
    v1 = b1[...]
    v2 = b2[...]
    s = v0 * v0 + v1 * v1 + v2 * v2
    o_ref[...] = s > _TH2


def kernel(t, xi):
    B, D = xi.shape
    NT = B // 128
    v = jnp.reshape(jnp.transpose(xi), (2, 8, NT, 128))
    v = jnp.transpose(v, (0, 2, 1, 3))
    v = jnp.reshape(v, (2 * NT, 8, 128))

    sc_i32 = _make_sc_kernel(_K_SC)(v)

    NTC = NT - _K_SC
    BLK = 1024
    tc_out = pl.pallas_call(
        _tc_body,
        grid=(NTC // BLK,),
        in_specs=[pl.BlockSpec(memory_space=pl.ANY)],
        out_specs=pl.BlockSpec((BLK, 128), lambda i: (i, 0)),
        out_shape=jax.ShapeDtypeStruct((NTC, 128), jnp.bool_),
        scratch_shapes=[
            pltpu.VMEM((BLK, 128), jnp.float32),
            pltpu.VMEM((BLK, 128), jnp.float32),
            pltpu.VMEM((BLK, 128), jnp.float32),
            pltpu.SemaphoreType.DMA,
        ],
    )(v)

    return jnp.concatenate(
        [sc_i32.astype(jnp.bool_), jnp.reshape(tc_out, (NTC * 128,))]
    )


# speedup vs baseline: 1.0656x; 1.0656x over previous
import functools
import math

import jax
import jax.numpy as jnp
from jax import lax
from jax.experimental import pallas as pl
from jax.experimental.pallas import tpu as pltpu
from jax.experimental.pallas import tpu_sc as plsc

_TH2 = (0.75 * math.pi) ** 2
_K_SC = 4096


def _make_sc_kernel(K):
    info = plsc.get_sparse_core_info()
    NC, NS, L = info.num_cores, info.num_subcores, info.num_lanes
    NW = NC * NS
    tiles_per_w = K // NW
    CH = tiles_per_w
    mesh = plsc.VectorSubcoreMesh(core_axis_name="c", subcore_axis_name="s")

    @functools.partial(
        pl.kernel,
        out_type=jax.ShapeDtypeStruct((K * 128,), jnp.int32),
        mesh=mesh,
        scratch_types=[
            pltpu.VMEM((CH, 128), jnp.float32),
            pltpu.VMEM((CH, 128), jnp.float32),
            pltpu.VMEM((CH, 128), jnp.float32),
            pltpu.VMEM((CH * 128,), jnp.int32),
        ],
        compiler_params=pltpu.CompilerParams(needs_layout_passes=False),
    )
    def body(v_hbm, out_hbm, b0, b1, b2, obuf):
        wid = lax.axis_index("s") * NC + lax.axis_index("c")
        tc0 = wid * tiles_per_w
        pltpu.sync_copy(v_hbm.at[pl.ds(tc0, CH), 0, :], b0)
        pltpu.sync_copy(v_hbm.at[pl.ds(tc0, CH), 1, :], b1)
        pltpu.sync_copy(v_hbm.at[pl.ds(tc0, CH), 2, :], b2)

        @plsc.parallel_loop(0, CH * 8, unroll=8)
        def inner(i):
            j = i >> 3
            l0 = (i & 7) * L
            v0 = b0[j, pl.ds(l0, L)]
            v1 = b1[j, pl.ds(l0, L)]
            v2 = b2[j, pl.ds(l0, L)]
            s = v0 * v0 + v1 * v1 + v2 * v2
            obuf[pl.ds(i * L, L)] = (s > _TH2).astype(jnp.int32)

        pltpu.sync_copy(obuf, out_hbm.at[pl.ds(tc0 * 128, CH * 128)])

    return body


def _tc_body(v_hbm, o_ref, b0, b1, b2, sem):
    i = pl.program_id(0)
    blk = o_ref.shape[0]
    r0 = _K_SC + i * blk
    c0 = pltpu.make_async_copy(v_hbm.at[pl.ds(r0, blk), 0, :], b0, sem)
    c1 = pltpu.make_async_copy(v_hbm.at[pl.ds(r0, blk), 1, :], b1, sem)
    c2 = pltpu.make_async_copy(v_hbm.at[pl.ds(r0, blk), 2, :], b2, sem)
    c0.start()
    c1.start()
    c2.start()
    c0.wait()
    c1.wait()
    c2.wait()
    v0 = b0[...]
    v1 = b1[...]
    v2 = b2[...]
    s = v0 * v0 + v1 * v1 + v2 * v2
    o_ref[...] = s > _TH2


def kernel(t, xi):
    B, D = xi.shape
    NT = B // 128
    v = jnp.reshape(jnp.transpose(xi), (2, 8, NT, 128))
    v = jnp.transpose(v, (0, 2, 1, 3))
    v = jnp.reshape(v, (2 * NT, 8, 128))

    sc_i32 = _make_sc_kernel(_K_SC)(v)

    NTC = NT - _K_SC
    BLK = 1024
    tc_out = pl.pallas_call(
        _tc_body,
        grid=(NTC // BLK,),
        in_specs=[pl.BlockSpec(memory_space=pl.ANY)],
        out_specs=pl.BlockSpec((BLK, 128), lambda i: (_K_SC // BLK + i, 0)),
        out_shape=jax.ShapeDtypeStruct((NT, 128), jnp.bool_),
        scratch_shapes=[
            pltpu.VMEM((BLK, 128), jnp.float32),
            pltpu.VMEM((BLK, 128), jnp.float32),
            pltpu.VMEM((BLK, 128), jnp.float32),
            pltpu.SemaphoreType.DMA,
        ],
    )(v)

    return lax.dynamic_update_slice(
        jnp.reshape(tc_out, (B,)), sc_i32.astype(jnp.bool_), (0,)
    )
